# Initial kernel scaffold; baseline (speedup 1.0000x reference)
import jax
import jax.numpy as jnp
from jax.experimental import pallas as pl


def kernel(x, batch, W, b):
    raise NotImplementedError("write your pallas kernel here")


# speedup vs baseline: 4.4589x; 4.4589x over previous
import jax
import jax.numpy as jnp
from jax import lax
from jax.experimental import pallas as pl
from jax.experimental.pallas import tpu as pltpu
from jax.experimental.pallas import tpu_sc as plsc

N = 100000
D = 128
NUM_GRAPHS = 64
NUM_CLASS = 2

NC = 2
NS = 16
NW = NC * NS

CH = 128
FULL_CHUNKS = N // CH
TAIL = N - FULL_CHUNKS * CH
K_ITERS = (FULL_CHUNKS + NW - 1) // NW
SEG_PAD = 72


def _sc_body(x_hbm, batch_hbm, part_out, cnt_out,
             xbuf, ibuf, ones, zbuf, acc_sh, cnt_sh):
  sid = lax.axis_index("s")
  cid = lax.axis_index("c")
  wid = sid * NC + cid

  zero16 = jnp.zeros((16,), jnp.float32)
  one16 = jnp.ones((16,), jnp.float32)

  def init_row(r, _):
    for v in range(D // 16):
      zbuf[r, pl.ds(16 * v, 16)] = zero16
      ones[r, pl.ds(16 * v, 16)] = one16
    return 0
  lax.fori_loop(0, SEG_PAD, init_row, 0)

  def init_ones(r, _):
    for v in range(D // 16):
      ones[r, pl.ds(16 * v, 16)] = one16
    return 0
  lax.fori_loop(SEG_PAD, CH, init_ones, 0)

  seg_base = sid * SEG_PAD
  pltpu.sync_copy(zbuf, acc_sh.at[pl.ds(seg_base, SEG_PAD)])

  @pl.when(sid == 0)
  def _():
    pltpu.sync_copy(zbuf, cnt_sh)

  plsc.subcore_barrier()

  def bias_idx(buf):
    for j in range(CH // 16):
      sl = pl.ds(16 * j, 16)
      ibuf[buf, sl] = ibuf[buf, sl] + seg_base

  def chunk_body(k, _):
    chunk = wid + NW * k

    @pl.when(chunk < FULL_CHUNKS)
    def _():
      base = chunk * CH
      pltpu.sync_copy(x_hbm.at[pl.ds(base, CH)], xbuf.at[0])
      pltpu.sync_copy(batch_hbm.at[pl.ds(base, CH)], ibuf.at[0])
      pltpu.sync_copy(ones, cnt_sh.at[ibuf.at[0]], add=True)
      bias_idx(0)
      pltpu.sync_copy(xbuf.at[0], acc_sh.at[ibuf.at[0]], add=True)
    return 0

  lax.fori_loop(0, K_ITERS, chunk_body, 0)

  @pl.when(wid == NW - 1)
  def _():
    base = FULL_CHUNKS * CH
    pltpu.sync_copy(x_hbm.at[pl.ds(base, TAIL)], xbuf.at[0, pl.ds(0, TAIL)])
    pltpu.sync_copy(batch_hbm.at[pl.ds(base, TAIL)], ibuf.at[0, pl.ds(0, TAIL)])
    dummy = jnp.full((16,), NUM_GRAPHS, jnp.int32)
    for j in range((CH - TAIL) // 16):
      ibuf[0, pl.ds(TAIL + 16 * j, 16)] = dummy
    pltpu.sync_copy(ones, cnt_sh.at[ibuf.at[0]], add=True)
    bias_idx(0)
    pltpu.sync_copy(xbuf.at[0], acc_sh.at[ibuf.at[0]], add=True)

  pltpu.sync_copy(acc_sh.at[pl.ds(seg_base, SEG_PAD)], part_out.at[wid])
  plsc.subcore_barrier()

  @pl.when(sid == 0)
  def _():
    pltpu.sync_copy(cnt_sh, cnt_out.at[cid])


@jax.jit
def _sc_segment_sums(x, batch32):
  mesh = plsc.VectorSubcoreMesh(core_axis_name="c", subcore_axis_name="s")
  f = pl.kernel(
      _sc_body,
      out_type=(
          jax.ShapeDtypeStruct((NW, SEG_PAD, D), jnp.float32),
          jax.ShapeDtypeStruct((NC, SEG_PAD, D), jnp.float32),
      ),
      mesh=mesh,
      scratch_types=[
          pltpu.VMEM((2, CH, D), jnp.float32),
          pltpu.VMEM((2, CH), jnp.int32),
          pltpu.VMEM((CH, D), jnp.float32),
          pltpu.VMEM((SEG_PAD, D), jnp.float32),
          pltpu.VMEM_SHARED((NS * SEG_PAD, D), jnp.float32),
          pltpu.VMEM_SHARED((SEG_PAD, D), jnp.float32),
      ],
  )
  return f(x, batch32)


def _tc_body(part_ref, cnt_ref, w_ref, b_ref, out_ref):
  sums = jnp.sum(part_ref[...], axis=0)[:NUM_GRAPHS]
  counts = jnp.sum(cnt_ref[...], axis=0)[:NUM_GRAPHS, 0]
  pooled = sums / jnp.maximum(counts, 1.0)[:, None]
  out_ref[...] = (
      jnp.dot(pooled, w_ref[...].T, preferred_element_type=jnp.float32)
      + b_ref[...]
  )


@jax.jit
def _tc_finish(part, cnt, W, b2d):
  return pl.pallas_call(
      _tc_body,
      out_shape=jax.ShapeDtypeStruct((NUM_GRAPHS, NUM_CLASS), jnp.float32),
  )(part, cnt, W, b2d)


def kernel(x, batch, W, b):
  batch32 = batch.astype(jnp.int32)
  part, cnt = _sc_segment_sums(x, batch32)
  return _tc_finish(part, cnt, W, b.reshape(1, NUM_CLASS))


# speedup vs baseline: 6.1003x; 1.3681x over previous
import jax
import jax.numpy as jnp
from jax import lax
from jax.experimental import pallas as pl
from jax.experimental.pallas import tpu as pltpu
from jax.experimental.pallas import tpu_sc as plsc

N = 100000
D = 128
NUM_GRAPHS = 64
NUM_CLASS = 2

NC = 2
NS = 16
NW = NC * NS

CH = 128
GRP = 4
GROUPS = 195
FULL_CHUNKS = N // CH
TAIL = N - FULL_CHUNKS * CH
K_ITERS = (GROUPS + NW - 1) // NW
SEG_PAD = 72
NPAD = 784 * 128


def _sc_body(x_hbm, batch_hbm, part_out, xbuf, ibuf, zbuf, acc_sh):
  sid = lax.axis_index("s")
  cid = lax.axis_index("c")
  wid = sid * NC + cid

  zero16 = jnp.zeros((16,), jnp.float32)

  @pl.when(sid == 0)
  def _():
    def init_row(r, _):
      for v in range(D // 16):
        zbuf[r, pl.ds(16 * v, 16)] = zero16
      return 0
    lax.fori_loop(0, SEG_PAD, init_row, 0)
    pltpu.sync_copy(zbuf, acc_sh)

  plsc.subcore_barrier()

  def group_body(k, _):
    group = wid + NW * k

    @pl.when(group < GROUPS)
    def _():
      g_base = group * (GRP * CH)
      pltpu.sync_copy(x_hbm.at[pl.ds(g_base, GRP * CH)], xbuf)
      for c in range(GRP):
        pltpu.sync_copy(batch_hbm.at[pl.ds(g_base + c * CH, CH)],
                        ibuf.at[c])
      for c in range(GRP):
        pltpu.sync_copy(xbuf.at[pl.ds(c * CH, CH)],
                        acc_sh.at[ibuf.at[c]], add=True)
    return 0

  lax.fori_loop(0, K_ITERS, group_body, 0)

  @pl.when(wid == NW - 1)
  def _():
    base = GROUPS * (GRP * CH)
    pltpu.sync_copy(x_hbm.at[pl.ds(base, CH)], xbuf.at[pl.ds(0, CH)])
    pltpu.sync_copy(batch_hbm.at[pl.ds(base, CH)], ibuf.at[0])
    pltpu.sync_copy(xbuf.at[pl.ds(0, CH)], acc_sh.at[ibuf.at[0]], add=True)

    tbase = FULL_CHUNKS * CH
    pltpu.sync_copy(x_hbm.at[pl.ds(tbase, TAIL)], xbuf.at[pl.ds(0, TAIL)])
    pltpu.sync_copy(batch_hbm.at[pl.ds(tbase, TAIL)],
                    ibuf.at[0, pl.ds(0, TAIL)])
    dummy = jnp.full((16,), NUM_GRAPHS, jnp.int32)
    for j in range((CH - TAIL) // 16):
      ibuf[0, pl.ds(TAIL + 16 * j, 16)] = dummy
    pltpu.sync_copy(xbuf.at[pl.ds(0, CH)], acc_sh.at[ibuf.at[0]], add=True)

  plsc.subcore_barrier()

  @pl.when(sid == 0)
  def _():
    pltpu.sync_copy(acc_sh, part_out.at[cid])


@jax.jit
def _sc_segment_sums(x, batch32):
  mesh = plsc.VectorSubcoreMesh(core_axis_name="c", subcore_axis_name="s")
  f = pl.kernel(
      _sc_body,
      out_type=jax.ShapeDtypeStruct((NC, SEG_PAD, D), jnp.float32),
      mesh=mesh,
      scratch_types=[
          pltpu.VMEM((GRP * CH, D), jnp.float32),
          pltpu.VMEM((GRP, CH), jnp.int32),
          pltpu.VMEM((SEG_PAD, D), jnp.float32),
          pltpu.VMEM_SHARED((SEG_PAD, D), jnp.float32),
      ],
  )
  return f(x, batch32)


def _tc_counts_body(b2_ref, out_ref):
  b2 = b2_ref[...]
  for g in range(NUM_GRAPHS):
    s = jnp.sum((b2 == g).astype(jnp.float32))
    out_ref[pl.ds(g, 1), :] = jnp.zeros((1, D), jnp.float32) + s


@jax.jit
def _tc_counts(b2):
  return pl.pallas_call(
      _tc_counts_body,
      out_shape=jax.ShapeDtypeStruct((NUM_GRAPHS, D), jnp.float32),
  )(b2)


def _tc_body(part_ref, cnt_ref, w_ref, b_ref, out_ref):
  sums = jnp.sum(part_ref[...], axis=0)[:NUM_GRAPHS]
  counts = cnt_ref[...][:, 0]
  pooled = sums / jnp.maximum(counts, 1.0)[:, None]
  out_ref[...] = (
      jnp.dot(pooled, w_ref[...].T, preferred_element_type=jnp.float32)
      + b_ref[...]
  )


@jax.jit
def _tc_finish(part, cnt, W, b2d):
  return pl.pallas_call(
      _tc_body,
      out_shape=jax.ShapeDtypeStruct((NUM_GRAPHS, NUM_CLASS), jnp.float32),
  )(part, cnt, W, b2d)


def kernel(x, batch, W, b):
  batch32 = batch.astype(jnp.int32)
  part = _sc_segment_sums(x, batch32)
  bpad = jnp.pad(batch32, (0, NPAD - N),
                 constant_values=NUM_GRAPHS).reshape(-1, 128)
  cnt = _tc_counts(bpad)
  return _tc_finish(part, cnt, W, b.reshape(1, NUM_CLASS))


# speedup vs baseline: 8.3015x; 1.3608x over previous
import jax
import jax.numpy as jnp
from jax import lax
from jax.experimental import pallas as pl
from jax.experimental.pallas import tpu as pltpu
from jax.experimental.pallas import tpu_sc as plsc

N = 100000
D = 128
NUM_GRAPHS = 64
NUM_CLASS = 2

NC = 2
NS = 16
NW = NC * NS
NV = D // 16

CH = 128
GRP = 3
GROUPS = 260
FULL_CHUNKS = N // CH
TAIL = N - FULL_CHUNKS * CH
K_ITERS = (GROUPS + NW - 1) // NW
ACC_ROWS = 128
SEG_PAD = 72
NPAD = 784 * 128


def _sc_body(x_hbm, batch_hbm, part_out, xbuf, ibufA, ibufB, acc, idbuf,
             mixbuf, pos, acc_sh, xsemA, xsemB, isemA, isemB):
  sid = lax.axis_index("s")
  cid = lax.axis_index("c")
  wid = sid * NC + cid

  zero16 = jnp.zeros((16,), jnp.float32)
  lanes = lax.iota(jnp.int32, 16)

  def init_row(r, _):
    for v in range(NV):
      acc[r, pl.ds(16 * v, 16)] = zero16
    return 0
  lax.fori_loop(0, ACC_ROWS, init_row, 0)

  for j in range(CH // 16):
    vals = lanes + 16 * j
    idbuf[0, pl.ds(16 * j, 16)] = jnp.where(vals < SEG_PAD, vals,
                                            NUM_GRAPHS)
  pos[0] = 0

  @pl.when(sid == 0)
  def _():
    pltpu.sync_copy(acc.at[pl.ds(0, SEG_PAD)], acc_sh)

  plsc.subcore_barrier()

  def start_gather(g, b):
    xsem = xsemA if b == 0 else xsemB
    isem = isemA if b == 0 else isemB
    group = wid + NW * g

    @pl.when(group < GROUPS)
    def _():
      base = group * (GRP * CH)
      ibuf = ibufA if b == 0 else ibufB
      pltpu.async_copy(x_hbm.at[pl.ds(base, GRP * CH)], xbuf.at[b], xsem)
      pltpu.async_copy(batch_hbm.at[pl.ds(base, GRP * CH)], ibuf, isem)

  def sum_rows(b, lo, hi, unroll):
    def rbody(r, carry):
      return tuple(carry[v] + xbuf[b, r, pl.ds(16 * v, 16)]
                   for v in range(NV))
    return lax.fori_loop(lo, hi, rbody, (zero16,) * NV, unroll=unroll)

  def consume(g, b):
    xsem = xsemA if b == 0 else xsemB
    isem = isemA if b == 0 else isemB
    group = wid + NW * g

    @pl.when(group < GROUPS)
    def _():
      ibuf = ibufA if b == 0 else ibufB
      pltpu.make_async_copy(x_hbm.at[pl.ds(0, GRP * CH)], xbuf.at[b],
                            xsem).wait()
      pltpu.make_async_copy(batch_hbm.at[pl.ds(0, GRP * CH)], ibuf,
                            isem).wait()
      for c in range(GRP):
        smn = ibuf[pl.ds(c * CH, 16)][0]
        smx = ibuf[pl.ds(c * CH + CH - 16, 16)][15]

        @pl.when(smn == smx)
        def _():
          sm = sum_rows(b, c * CH, (c + 1) * CH, unroll=4)
          for v in range(NV):
            sl = pl.ds(16 * v, 16)
            acc[smn, sl] = acc[smn, sl] + sm[v]

        @pl.when(smn != smx)
        def _():
          cnt = pos[0]
          pos[cnt + 1] = group * (GRP * CH) + c * CH
          pos[0] = cnt + 1

  start_gather(0, 0)

  def pair_body(kk, _):
    g0 = 2 * kk
    start_gather(g0 + 1, 1)
    consume(g0, 0)
    start_gather(g0 + 2, 0)
    consume(g0 + 1, 1)
    return 0

  lax.fori_loop(0, (K_ITERS + 1) // 2, pair_body, 0)

  def mixed_body(m, _):
    base = pl.multiple_of(pos[m + 1], CH)
    pltpu.sync_copy(x_hbm.at[pl.ds(base, CH)], xbuf.at[0, pl.ds(0, CH)])
    pltpu.sync_copy(batch_hbm.at[pl.ds(base, CH)], mixbuf.at[0])
    pltpu.sync_copy(xbuf.at[0, pl.ds(0, CH)], acc_sh.at[mixbuf.at[0]],
                    add=True)
    return 0

  lax.fori_loop(0, pos[0], mixed_body, 0)

  @pl.when(wid == NW - 1)
  def _():
    base = GROUPS * (GRP * CH)
    pltpu.sync_copy(x_hbm.at[pl.ds(base, CH)], xbuf.at[0, pl.ds(0, CH)])
    pltpu.sync_copy(batch_hbm.at[pl.ds(base, CH)], mixbuf.at[0])
    pltpu.sync_copy(xbuf.at[0, pl.ds(0, CH)], acc_sh.at[mixbuf.at[0]],
                    add=True)

    tbase = FULL_CHUNKS * CH
    pltpu.sync_copy(x_hbm.at[pl.ds(tbase, TAIL)],
                    xbuf.at[0, pl.ds(0, TAIL)])
    pltpu.sync_copy(batch_hbm.at[pl.ds(tbase, TAIL)],
                    mixbuf.at[0, pl.ds(0, TAIL)])
    dummy = jnp.full((16,), NUM_GRAPHS, jnp.int32)
    for j in range((CH - TAIL) // 16):
      mixbuf[0, pl.ds(TAIL + 16 * j, 16)] = dummy
    pltpu.sync_copy(xbuf.at[0, pl.ds(0, CH)], acc_sh.at[mixbuf.at[0]],
                    add=True)

  pltpu.sync_copy(acc.at[pl.ds(0, CH)], acc_sh.at[idbuf.at[0]], add=True)
  plsc.subcore_barrier()

  @pl.when(sid == 0)
  def _():
    pltpu.sync_copy(acc_sh, part_out.at[cid])


@jax.jit
def _sc_segment_sums(x, batch32):
  mesh = plsc.VectorSubcoreMesh(core_axis_name="c", subcore_axis_name="s")
  f = pl.kernel(
      _sc_body,
      out_type=jax.ShapeDtypeStruct((NC, SEG_PAD, D), jnp.float32),
      mesh=mesh,
      scratch_types=[
          pltpu.VMEM((2, GRP * CH, D), jnp.float32),
          pltpu.VMEM((GRP * CH,), jnp.int32),
          pltpu.VMEM((GRP * CH,), jnp.int32),
          pltpu.VMEM((ACC_ROWS, D), jnp.float32),
          pltpu.VMEM((8, CH), jnp.int32),
          pltpu.VMEM((8, CH), jnp.int32),
          pltpu.SMEM((40,), jnp.int32),
          pltpu.VMEM_SHARED((SEG_PAD, D), jnp.float32),
          pltpu.SemaphoreType.DMA,
          pltpu.SemaphoreType.DMA,
          pltpu.SemaphoreType.DMA,
          pltpu.SemaphoreType.DMA,
      ],
  )
  return f(x, batch32)


def _tc_counts_body(b2_ref, out_ref):
  b2 = b2_ref[...]
  for g in range(NUM_GRAPHS):
    s = jnp.sum((b2 == g).astype(jnp.float32))
    out_ref[pl.ds(g, 1), :] = jnp.zeros((1, D), jnp.float32) + s


@jax.jit
def _tc_counts(b2):
  return pl.pallas_call(
      _tc_counts_body,
      out_shape=jax.ShapeDtypeStruct((NUM_GRAPHS, D), jnp.float32),
  )(b2)


def _tc_body(part_ref, cnt_ref, w_ref, b_ref, out_ref):
  sums = jnp.sum(part_ref[...], axis=0)[:NUM_GRAPHS]
  counts = cnt_ref[...][:, 0]
  pooled = sums / jnp.maximum(counts, 1.0)[:, None]
  out_ref[...] = (
      jnp.dot(pooled, w_ref[...].T, preferred_element_type=jnp.float32)
      + b_ref[...]
  )


@jax.jit
def _tc_finish(part, cnt, W, b2d):
  return pl.pallas_call(
      _tc_body,
      out_shape=jax.ShapeDtypeStruct((NUM_GRAPHS, NUM_CLASS), jnp.float32),
  )(part, cnt, W, b2d)


def kernel(x, batch, W, b):
  batch32 = batch.astype(jnp.int32)
  part = _sc_segment_sums(x, batch32)
  bpad = jnp.pad(batch32, (0, NPAD - N),
                 constant_values=NUM_GRAPHS).reshape(-1, 128)
  cnt = _tc_counts(bpad)
  return _tc_finish(part, cnt, W, b.reshape(1, NUM_CLASS))
